# Initial kernel scaffold; baseline (speedup 1.0000x reference)
import jax
import jax.numpy as jnp
from jax.experimental import pallas as pl


def kernel(p2, p3, p4, p5, p6, proposals):
    raise NotImplementedError("write your pallas kernel here")


# speedup vs baseline: 19.3556x; 19.3556x over previous
import functools

import jax
import jax.numpy as jnp
from jax import lax
from jax.experimental import pallas as pl
from jax.experimental.pallas import tpu as pltpu
from jax.experimental.pallas import tpu_sc as plsc

NC, NS, L = 2, 16, 16
NW = NC * NS
B, R, C = 2, 1000, 96
RPAD = 1024
NBOX = B * RPAD
BOX_PER_W = NBOX // NW
PER_BATCH = 21824
CB = C // L


def _body(table, props, out, boxes_v, idx_v, rows_v, fvecs, out_v, sem):
    wid = lax.axis_index("s") * NC + lax.axis_index("c")
    base_box = wid * BOX_PER_W
    pltpu.sync_copy(props.at[pl.ds(base_box * 4, BOX_PER_W * 4)],
                    boxes_v.at[pl.ds(0, BOX_PER_W * 4)])
    lane = lax.iota(jnp.int32, 16)
    lanef = lane.astype(jnp.float32)
    msk14 = lane < 14

    def box_body(i, carry):
        g = base_box + i
        x1 = boxes_v[pl.ds(i * 4, 16)][0]
        y1 = boxes_v[pl.ds(i * 4 + 1, 16)][0]
        x2 = boxes_v[pl.ds(i * 4 + 2, 16)][0]
        y2 = boxes_v[pl.ds(i * 4 + 3, 16)][0]
        bw = jnp.maximum(x2 - x1, 1.0)
        bh = jnp.maximum(y2 - y1, 1.0)
        area = bw * bh
        ge3 = area >= 12544.0
        ge4 = area >= 50176.0
        ge5 = area >= 200704.0
        ge6 = area >= 802816.0
        scale = jnp.where(ge6, 0.015625,
                jnp.where(ge5, 0.03125,
                jnp.where(ge4, 0.0625,
                jnp.where(ge3, 0.125, 0.25))))
        wl = jnp.where(ge6, 8, jnp.where(ge5, 16, jnp.where(ge4, 32,
             jnp.where(ge3, 64, 128)))).astype(jnp.int32)
        lbase = jnp.where(ge6, 21760, jnp.where(ge5, 21504,
                jnp.where(ge4, 20480, jnp.where(ge3, 16384, 0)))).astype(jnp.int32)
        base = lbase + jnp.where(g >= RPAD, PER_BATCH, 0).astype(jnp.int32)
        wf = wl.astype(jnp.float32)

        x1s = x1 * scale
        y1s = y1 * scale
        x2s = x2 * scale
        y2s = y2 * scale
        bin_w = jnp.maximum(x2s - x1s, 1.0) * (1.0 / 7.0)
        bin_h = jnp.maximum(y2s - y1s, 1.0) * (1.0 / 7.0)

        ys = y1s + (0.5 * lanef + 0.25) * bin_h
        xs = x1s + (0.5 * lanef + 0.25) * bin_w
        vy = jnp.where((ys > -1.0) & (ys < wf), 1.0, 0.0)
        vx = jnp.where((xs > -1.0) & (xs < wf), 1.0, 0.0)
        yc = jnp.clip(ys, 0.0, wf - 1.0)
        xc = jnp.clip(xs, 0.0, wf - 1.0)
        y0i = yc.astype(jnp.int32)
        x0i = xc.astype(jnp.int32)
        ly = yc - y0i.astype(jnp.float32)
        lx = xc - x0i.astype(jnp.float32)
        y1i = jnp.minimum(y0i + 1, wl - 1)
        x1i = jnp.minimum(x0i + 1, wl - 1)
        hyv = (1.0 - ly) * vy
        lyv = ly * vy
        hxv = (1.0 - lx) * vx
        lxv = lx * vx
        rt = base + y0i * wl
        rb = base + y1i * wl
        fvecs[pl.ds(0, 16)] = hyv
        fvecs[pl.ds(16, 16)] = lyv

        for y in range(14):
            cy = y // 2
            off = (y % 2) * 64
            rt_s = rt[y]
            rb_s = rb[y]
            idx_v[cy, pl.ds(off, 16)] = x0i + rt_s
            idx_v[cy, pl.ds(off + 16, 16)] = x1i + rt_s
            idx_v[cy, pl.ds(off + 32, 16)] = x0i + rb_s
            idx_v[cy, pl.ds(off + 48, 16)] = x1i + rb_s

        handles = [
            pltpu.async_copy(table.at[idx_v.at[c]], rows_v.at[c], sem)
            for c in range(7)
        ]
        for h in handles:
            h.wait()

        def chunk_body(cc, carry2):
            hy0 = fvecs[pl.ds(2 * cc, 16)][0]
            ly0 = fvecs[pl.ds(16 + 2 * cc, 16)][0]
            hy1 = fvecs[pl.ds(2 * cc + 1, 16)][0]
            ly1 = fvecs[pl.ds(16 + 2 * cc + 1, 16)][0]
            hys = (hy0, hy1)
            lys = (ly0, ly1)
            for ox in range(7):
                acc = [jnp.zeros((16,), jnp.float32) for _ in range(CB)]
                for sy in range(2):
                    hy_s = hys[sy]
                    ly_s = lys[sy]
                    for sx in range(2):
                        xj = 2 * ox + sx
                        hx_s = hxv[xj]
                        lx_s = lxv[xj]
                        w00 = hy_s * hx_s
                        w01 = hy_s * lx_s
                        w10 = ly_s * hx_s
                        w11 = ly_s * lx_s
                        p = sy * 64 + xj
                        for k in range(CB):
                            sl = pl.ds(k * 16, 16)
                            acc[k] = (acc[k]
                                      + w00 * rows_v[cc, p, sl]
                                      + w01 * rows_v[cc, p + 16, sl]
                                      + w10 * rows_v[cc, p + 32, sl]
                                      + w11 * rows_v[cc, p + 48, sl])
                obase = (cc * 7 + ox) * 96
                for k in range(CB):
                    out_v[pl.ds(obase + k * 16, 16)] = acc[k] * 0.25
            return carry2

        lax.fori_loop(0, 7, chunk_body, 0, unroll=False)
        pltpu.sync_copy(out_v, out.at[g])
        return carry

    lax.fori_loop(0, BOX_PER_W, box_body, 0, unroll=False)


@functools.partial(
    pl.kernel,
    mesh=plsc.VectorSubcoreMesh(core_axis_name="c", subcore_axis_name="s"),
    out_type=jax.ShapeDtypeStruct((NBOX, 7 * 7 * C), jnp.float32),
    scratch_types=[
        pltpu.VMEM((BOX_PER_W * 4 + 16,), jnp.float32),
        pltpu.VMEM((7, 128), jnp.int32),
        pltpu.VMEM((7, 128, C), jnp.float32),
        pltpu.VMEM((48,), jnp.float32),
        pltpu.VMEM((7 * 7 * C,), jnp.float32),
        pltpu.SemaphoreType.DMA,
    ],
    compiler_params=pltpu.CompilerParams(use_tc_tiling_on_sc=False),
)
def _roi_kernel(table, props, out, boxes_v, idx_v, rows_v, fvecs, out_v, sem):
    _body(table, props, out, boxes_v, idx_v, rows_v, fvecs, out_v, sem)


def kernel(p2, p3, p4, p5, p6, proposals):
    table = jnp.concatenate(
        [p.reshape(B, -1, C) for p in (p2, p3, p4, p5, p6)], axis=1
    ).reshape(B * PER_BATCH, C)
    props = jnp.zeros((B, RPAD, 4), jnp.float32).at[:, :R].set(proposals)
    props = props.reshape(NBOX * 4)
    out = _roi_kernel(table, props)
    return out.reshape(B, RPAD, 7, 7, C)[:, :R]
